# baseline (speedup 1.0000x reference)
import jax
import jax.numpy as jnp
from jax.experimental import pallas as pl
from jax.experimental.pallas import tpu as pltpu

_ALPHA = 0.3


def _split(a):
    hi = a.astype(jnp.bfloat16)
    lo = (a - hi.astype(jnp.float32)).astype(jnp.bfloat16)
    return hi, lo


def _dot_bf16(a, b):
    return jax.lax.dot_general(a, b, (((1,), (0,)), ((), ())),
                               preferred_element_type=jnp.float32)


def _dot3(a_split, b_split):
    a_hi, a_lo = a_split
    b_hi, b_lo = b_split
    return (_dot_bf16(a_hi, b_hi) + _dot_bf16(a_hi, b_lo)
            + _dot_bf16(a_lo, b_hi))


def _fwd_kernel(feat_ref, w1_ref, b1_ref, w2_ref, b2_ref, w3_ref, b3_ref,
                dw_ref, db_ref, ow_ref, ob_ref, out_ref):
    f32 = jnp.float32
    hi_prec = jax.lax.Precision.HIGHEST
    n = feat_ref.shape[1]

    ri = jax.lax.broadcasted_iota(jnp.int32, (n, n), 0)
    ci = jax.lax.broadcasted_iota(jnp.int32, (n, n), 1)
    lower = (ri > ci).astype(f32)
    eye = (ri == ci).astype(f32)

    w1 = w1_ref[...]
    w2 = w2_ref[...]
    w3 = w3_ref[...]
    b1 = b1_ref[...]
    b2 = b2_ref[...]
    b3 = b3_ref[...]
    dw = dw_ref[...]
    db = db_ref[...]
    ow = ow_ref[...]
    ob = ob_ref[...]

    def layer(a_split, x, w, bvec):
        agg = _dot3(a_split, _split(x))
        h = _ALPHA * x + (1.0 - _ALPHA) * agg
        z = jnp.dot(h, w, preferred_element_type=f32) + bvec
        return jnp.tanh(z)

    x0 = feat_ref[0]
    r2 = jnp.sum(x0 * x0, axis=1, keepdims=True)
    g = jax.lax.dot_general(x0, x0, (((1,), (1,)), ((), ())),
                            precision=hi_prec, preferred_element_type=f32)
    d2 = r2 + jnp.transpose(r2) - 2.0 * g
    dist = jnp.sqrt(jnp.maximum(d2, 1e-12))
    dist_l = dist * lower
    deg = jnp.sum(dist_l, axis=1, keepdims=True) + 1.0
    dinv = jax.lax.rsqrt(deg)
    a_mat = (dinv * jnp.transpose(dinv)) * (dist_l + eye)
    a_split = _split(a_mat)

    x1 = layer(a_split, x0, w1, b1)
    x2 = layer(a_split, x1, w2, b2)
    x3 = layer(a_split, x2, w3, b3)

    pooled = jnp.mean(x3, axis=0, keepdims=True)
    h = jnp.tanh(jnp.dot(pooled, dw, preferred_element_type=f32) + db)
    out = jnp.dot(h, ow, preferred_element_type=f32) + ob
    out_ref[...] = out[None]


def kernel(features, conv1_W, conv1_b, conv2_W, conv2_b, conv3_W, conv3_b,
           dense_W, dense_b, out_W, out_b):
    nb, n, hid = features.shape
    rep = lambda *_: (0, 0)
    out = pl.pallas_call(
        _fwd_kernel,
        grid=(nb,),
        in_specs=[
            pl.BlockSpec((1, n, hid), lambda b: (b, 0, 0)),
            pl.BlockSpec(conv1_W.shape, rep),
            pl.BlockSpec((1, 2 * hid), rep),
            pl.BlockSpec(conv2_W.shape, rep),
            pl.BlockSpec((1, 2 * hid), rep),
            pl.BlockSpec(conv3_W.shape, rep),
            pl.BlockSpec((1, 2 * hid), rep),
            pl.BlockSpec(dense_W.shape, rep),
            pl.BlockSpec((1, hid), rep),
            pl.BlockSpec(out_W.shape, rep),
            pl.BlockSpec((1, 2), rep),
        ],
        out_specs=pl.BlockSpec((1, 1, 2), lambda b: (b, 0, 0)),
        out_shape=jax.ShapeDtypeStruct((nb, 1, 2), jnp.float32),
        compiler_params=pltpu.CompilerParams(
            dimension_semantics=("parallel",)),
    )(features,
      conv1_W, conv1_b.reshape(1, -1),
      conv2_W, conv2_b.reshape(1, -1),
      conv3_W, conv3_b.reshape(1, -1),
      dense_W, dense_b.reshape(1, -1),
      out_W, out_b.reshape(1, -1))
    return out.reshape(nb, 2)


# speedup vs baseline: 1.4976x; 1.4976x over previous
import jax
import jax.numpy as jnp
from jax.experimental import pallas as pl

_ALPHA = 0.3


def _split(a):
    hi = a.astype(jnp.bfloat16)
    lo = (a - hi.astype(jnp.float32)).astype(jnp.bfloat16)
    return hi, lo


def _dot_bf16(a, b):
    return jax.lax.dot_general(a, b, (((1,), (0,)), ((), ())),
                               preferred_element_type=jnp.float32)


def _dot3(a_split, b_split):
    a_hi, a_lo = a_split
    b_hi, b_lo = b_split
    return (_dot_bf16(a_hi, b_hi) + _dot_bf16(a_hi, b_lo)
            + _dot_bf16(a_lo, b_hi))


def _fwd_kernel(feat_ref, w1_ref, b1_ref, w2_ref, b2_ref, w3_ref, b3_ref,
                dw_ref, db_ref, ow_ref, ob_ref, out_ref):
    f32 = jnp.float32
    hi_prec = jax.lax.Precision.HIGHEST
    nb, n, _ = feat_ref.shape

    ri = jax.lax.broadcasted_iota(jnp.int32, (n, n), 0)
    ci = jax.lax.broadcasted_iota(jnp.int32, (n, n), 1)
    lower = (ri > ci).astype(f32)
    eye = (ri == ci).astype(f32)

    w1 = w1_ref[...]
    w2 = w2_ref[...]
    w3 = w3_ref[...]
    b1 = b1_ref[...]
    b2 = b2_ref[...]
    b3 = b3_ref[...]
    dw = dw_ref[...]
    db = db_ref[...]
    ow = ow_ref[...]
    ob = ob_ref[...]

    a_splits = []
    for b in range(nb):
        x0 = feat_ref[b]
        r2 = jnp.sum(x0 * x0, axis=1, keepdims=True)
        g = jax.lax.dot_general(x0, x0, (((1,), (1,)), ((), ())),
                                precision=hi_prec, preferred_element_type=f32)
        d2 = r2 + jnp.transpose(r2) - 2.0 * g
        dist = jnp.sqrt(jnp.maximum(d2, 1e-12))
        dist_l = dist * lower
        deg = jnp.sum(dist_l, axis=1, keepdims=True) + 1.0
        dinv = jax.lax.rsqrt(deg)
        a_splits.append(_split((dinv * jnp.transpose(dinv)) * (dist_l + eye)))

    def layer(x_all, w, bvec):
        aggs = [_dot3(a_splits[b], _split(x_all[b * n:(b + 1) * n]))
                for b in range(nb)]
        h = _ALPHA * x_all + (1.0 - _ALPHA) * jnp.concatenate(aggs, axis=0)
        z = jnp.dot(h, w, preferred_element_type=f32) + bvec
        return jnp.tanh(z)

    x_all = feat_ref[...].reshape(nb * n, -1)
    x_all = layer(x_all, w1, b1)
    x_all = layer(x_all, w2, b2)
    x_all = layer(x_all, w3, b3)

    pooled = jnp.mean(x_all.reshape(nb, n, -1), axis=1)
    h = jnp.tanh(jnp.dot(pooled, dw, preferred_element_type=f32) + db)
    out_ref[...] = jnp.dot(h, ow, preferred_element_type=f32) + ob


def kernel(features, conv1_W, conv1_b, conv2_W, conv2_b, conv3_W, conv3_b,
           dense_W, dense_b, out_W, out_b):
    nb = features.shape[0]
    return pl.pallas_call(
        _fwd_kernel,
        out_shape=jax.ShapeDtypeStruct((nb, 2), jnp.float32),
    )(features,
      conv1_W, conv1_b.reshape(1, -1),
      conv2_W, conv2_b.reshape(1, -1),
      conv3_W, conv3_b.reshape(1, -1),
      dense_W, dense_b.reshape(1, -1),
      out_W, out_b.reshape(1, -1))
